# baseline (speedup 1.0000x reference)
import functools

import jax
import jax.numpy as jnp
from jax import lax
from jax.experimental import pallas as pl
from jax.experimental.pallas import tpu as pltpu
from jax.experimental.pallas import tpu_sc as plsc

N = 10000
E = 320000
D = 128

NC = 2
NS = 16
NW = NC * NS
CH = 88
K = 120
G = 8
NGROUPS = K // G
E_PAD = NW * K * CH
ACC_ROWS = 10240
ROWS_PER_TILE = ACC_ROWS // NS


def _make_sc_aggregate():
    mesh = plsc.VectorSubcoreMesh(core_axis_name="c", subcore_axis_name="s")

    @functools.partial(
        pl.kernel,
        mesh=mesh,
        out_type=jax.ShapeDtypeStruct((NC, ACC_ROWS, D), jnp.float32),
        scratch_types=[
            pltpu.VMEM((2 * G, CH), jnp.int32),
            pltpu.VMEM((2 * G, CH), jnp.int32),
            pltpu.VMEM((4, CH, D), jnp.float32),
            pltpu.VMEM_SHARED((ACC_ROWS, D), jnp.float32),
            [pltpu.SemaphoreType.DMA] * 4,
            [pltpu.SemaphoreType.DMA] * 4,
            pltpu.SemaphoreType.DMA,
            pltpu.SemaphoreType.DMA,
        ],
    )
    def sc_agg(src_hbm, rows_hbm, cols_hbm, zeros_hbm, part_hbm,
               rows_v, cols_v, bufs, acc, gsems, ssems, irsem, icsem):
        c = lax.axis_index("c")
        s = lax.axis_index("s")
        wid = c * NS + s

        def fire_idx(g):
            r = (g % 2) * G
            pltpu.async_copy(rows_hbm.at[pl.ds(wid * K + g * G, G)],
                             rows_v.at[pl.ds(r, G)], irsem)
            pltpu.async_copy(cols_hbm.at[pl.ds(wid * K + g * G, G)],
                             cols_v.at[pl.ds(r, G)], icsem)

        def wait_idx(g):
            r = (g % 2) * G
            pltpu.make_async_copy(rows_hbm.at[pl.ds(wid * K + g * G, G)],
                                  rows_v.at[pl.ds(r, G)], irsem).wait()
            pltpu.make_async_copy(cols_hbm.at[pl.ds(wid * K + g * G, G)],
                                  cols_v.at[pl.ds(r, G)], icsem).wait()

        def fire_gather(ring_row, b):
            pltpu.async_copy(src_hbm.at[cols_v.at[ring_row]], bufs.at[b],
                             gsems[b])

        def wait_gather(b):
            pltpu.make_async_copy(src_hbm.at[cols_v.at[0]], bufs.at[b],
                                  gsems[b]).wait()

        def fire_scatter(ring_row, b):
            pltpu.async_copy(bufs.at[b], acc.at[rows_v.at[ring_row]],
                             ssems[b], add=True)

        def wait_scatter(b):
            pltpu.make_async_copy(bufs.at[b], acc.at[rows_v.at[0]],
                                  ssems[b]).wait()

        fire_idx(0)
        wait_idx(0)
        fire_gather(0, 0)
        fire_gather(1, 1)
        fire_gather(2, 2)
        pltpu.sync_copy(zeros_hbm, acc.at[pl.ds(s * ROWS_PER_TILE,
                                                ROWS_PER_TILE)])
        plsc.subcore_barrier()

        def block(m, carry):
            for q in range(G):
                u = q % 4
                u3 = (q + 3) % 4
                wait_gather(u)
                fire_scatter((m % 2) * G + q, u)
                if q == 0:
                    @pl.when(m >= 1)
                    def _():
                        wait_scatter(u3)
                    @pl.when(m + 1 < NGROUPS)
                    def _():
                        fire_idx(m + 1)
                else:
                    wait_scatter(u3)
                if q == 5:
                    @pl.when(m + 1 < NGROUPS)
                    def _():
                        wait_idx(m + 1)
                gg = m + (q + 3) // G
                grow3 = (gg % 2) * G + (q + 3) % G
                if q + 3 < G:
                    fire_gather(grow3, u3)
                else:
                    @pl.when(m + 1 < NGROUPS)
                    def _():
                        fire_gather(grow3, u3)
            return carry
        lax.fori_loop(0, NGROUPS, block, 0)
        wait_scatter((K - 1) % 4)
        plsc.subcore_barrier()

        pltpu.sync_copy(
            acc.at[pl.ds(s * ROWS_PER_TILE, ROWS_PER_TILE)],
            part_hbm.at[c, pl.ds(s * ROWS_PER_TILE, ROWS_PER_TILE)])

    return sc_agg


_sc_aggregate = _make_sc_aggregate()

_NBLK = 10
_BLK = N // _NBLK


def _tc1_body(pa_ref, pb_ref, w_ref, b_ref, o_ref):
    agg = pa_ref[...] + pb_ref[...]
    o_ref[...] = jnp.maximum(
        jnp.dot(agg, w_ref[...], preferred_element_type=jnp.float32)
        + b_ref[...], 0.0)


def _tc2_body(pa_ref, pb_ref, w1_ref, b1_ref, wp_ref, bp_ref, o_ref):
    agg = pa_ref[...] + pb_ref[...]
    t = jnp.maximum(
        jnp.dot(agg, w1_ref[...], preferred_element_type=jnp.float32)
        + b1_ref[...], 0.0) + agg
    o_ref[...] = (jnp.dot(t, wp_ref[...], preferred_element_type=jnp.float32)
                  + bp_ref[...])


_p_spec = pl.BlockSpec((_BLK, D), lambda i: (i, 0))

_tc1 = pl.pallas_call(
    _tc1_body,
    grid=(_NBLK,),
    in_specs=[
        _p_spec,
        _p_spec,
        pl.BlockSpec((D, D), lambda i: (0, 0)),
        pl.BlockSpec((1, D), lambda i: (0, 0)),
    ],
    out_specs=pl.BlockSpec((_BLK, D), lambda i: (i, 0)),
    out_shape=jax.ShapeDtypeStruct((N, D), jnp.float32),
)

_tc2 = pl.pallas_call(
    _tc2_body,
    grid=(_NBLK,),
    in_specs=[
        _p_spec,
        _p_spec,
        pl.BlockSpec((D, D), lambda i: (0, 0)),
        pl.BlockSpec((1, D), lambda i: (0, 0)),
        pl.BlockSpec((D, D), lambda i: (0, 0)),
        pl.BlockSpec((1, D), lambda i: (0, 0)),
    ],
    out_specs=pl.BlockSpec((_BLK, D), lambda i: (i, 0)),
    out_shape=jax.ShapeDtypeStruct((N, D), jnp.float32),
)


def kernel(x, edge_index, W0, b0, W1, b1, Wp, bp):
    ei = edge_index.astype(jnp.int32)
    npad = E_PAD - E
    pad_dst = N + (jnp.arange(npad, dtype=jnp.int32) % (ACC_ROWS - N))
    pad_src = jnp.arange(npad, dtype=jnp.int32) % N
    rows = jnp.concatenate([ei[0], pad_dst]).reshape(E_PAD // CH, CH)
    cols = jnp.concatenate([ei[1], pad_src]).reshape(E_PAD // CH, CH)
    zeros = jnp.zeros((ROWS_PER_TILE, D), jnp.float32)

    p0 = _sc_aggregate(x, rows, cols, zeros)
    h = _tc1(p0[0], p0[1], W0, b0.reshape(1, D))
    p1 = _sc_aggregate(h, rows, cols, zeros)
    return _tc2(p1[0], p1[1], W1, b1.reshape(1, D), Wp, bp.reshape(1, D))


# speedup vs baseline: 1.0771x; 1.0771x over previous
import functools

import jax
import jax.numpy as jnp
from jax import lax
from jax.experimental import pallas as pl
from jax.experimental.pallas import tpu as pltpu
from jax.experimental.pallas import tpu_sc as plsc

N = 10000
E = 320000
D = 128

NC = 2
NS = 16
NW = NC * NS
CH = 112
K = 96
G = 8
NGROUPS = K // G
UNROLL = 24
E_PAD = NW * K * CH
ACC_ROWS = 10240
ROWS_PER_TILE = ACC_ROWS // NS


def _make_sc_aggregate():
    mesh = plsc.VectorSubcoreMesh(core_axis_name="c", subcore_axis_name="s")

    @functools.partial(
        pl.kernel,
        mesh=mesh,
        out_type=(jax.ShapeDtypeStruct((ACC_ROWS, D), jnp.float32),
                  jax.ShapeDtypeStruct((ACC_ROWS, D), jnp.float32)),
        scratch_types=[
            pltpu.VMEM((2 * G, CH), jnp.int32),
            pltpu.VMEM((2 * G, CH), jnp.int32),
            pltpu.VMEM((3, CH, D), jnp.float32),
            pltpu.VMEM_SHARED((ACC_ROWS, D), jnp.float32),
            [pltpu.SemaphoreType.DMA] * 3,
            [pltpu.SemaphoreType.DMA] * 3,
            pltpu.SemaphoreType.DMA,
            pltpu.SemaphoreType.DMA,
        ],
    )
    def sc_agg(src_hbm, rows_hbm, cols_hbm, zeros_hbm, part0_hbm, part1_hbm,
               rows_v, cols_v, bufs, acc, gsems, ssems, irsem, icsem):
        c = lax.axis_index("c")
        s = lax.axis_index("s")
        wid = c * NS + s

        def fire_idx(g):
            r = (g % 2) * G
            pltpu.async_copy(rows_hbm.at[pl.ds(wid * K + g * G, G)],
                             rows_v.at[pl.ds(r, G)], irsem)
            pltpu.async_copy(cols_hbm.at[pl.ds(wid * K + g * G, G)],
                             cols_v.at[pl.ds(r, G)], icsem)

        def wait_idx(g):
            r = (g % 2) * G
            pltpu.make_async_copy(rows_hbm.at[pl.ds(wid * K + g * G, G)],
                                  rows_v.at[pl.ds(r, G)], irsem).wait()
            pltpu.make_async_copy(cols_hbm.at[pl.ds(wid * K + g * G, G)],
                                  cols_v.at[pl.ds(r, G)], icsem).wait()

        def fire_gather(ring_row, b):
            pltpu.async_copy(src_hbm.at[cols_v.at[ring_row]], bufs.at[b],
                             gsems[b])

        def wait_gather(b):
            pltpu.make_async_copy(src_hbm.at[cols_v.at[0]], bufs.at[b],
                                  gsems[b]).wait()

        def fire_scatter(ring_row, b):
            pltpu.async_copy(bufs.at[b], acc.at[rows_v.at[ring_row]],
                             ssems[b], add=True)

        def wait_scatter(b):
            pltpu.make_async_copy(bufs.at[b], acc.at[rows_v.at[0]],
                                  ssems[b]).wait()

        fire_idx(0)
        wait_idx(0)
        fire_gather(0, 0)
        fire_gather(1, 1)
        pltpu.sync_copy(zeros_hbm, acc.at[pl.ds(s * ROWS_PER_TILE,
                                                ROWS_PER_TILE)])
        plsc.subcore_barrier()

        def block(m, carry):
            for q in range(UNROLL):
                t = m * UNROLL + q
                j = q // G
                g = 3 * m + j
                u = q % 3
                u2 = (q + 2) % 3
                grow = ((g % 2) * G + q % G)
                wait_gather(u)
                fire_scatter(grow, u)
                if q == 0:
                    @pl.when(m >= 1)
                    def _():
                        wait_scatter(u2)
                else:
                    wait_scatter(u2)
                if q % G == 1:
                    @pl.when(g + 1 < NGROUPS)
                    def _():
                        fire_idx(g + 1)
                if q % G == 6:
                    @pl.when(g + 1 < NGROUPS)
                    def _():
                        wait_idx(g + 1)
                g2 = 3 * m + (q + 2) // G
                grow2 = (g2 % 2) * G + (q + 2) % G
                if q < UNROLL - 2:
                    fire_gather(grow2, u2)
                else:
                    @pl.when(m + 1 < K // UNROLL)
                    def _():
                        fire_gather(grow2, u2)
            return carry
        lax.fori_loop(0, K // UNROLL, block, 0)
        wait_scatter((K - 1) % 3)
        plsc.subcore_barrier()

        sl = pl.ds(s * ROWS_PER_TILE, ROWS_PER_TILE)
        @pl.when(c == 0)
        def _():
            pltpu.sync_copy(acc.at[sl], part0_hbm.at[sl])
        @pl.when(c == 1)
        def _():
            pltpu.sync_copy(acc.at[sl], part1_hbm.at[sl])

    return sc_agg


_sc_aggregate = _make_sc_aggregate()

def _tc1_body(pa_ref, pb_ref, w_ref, b_ref, o_ref):
    agg = pa_ref[:N] + pb_ref[:N]
    o_ref[...] = jnp.maximum(
        jnp.dot(agg, w_ref[...], preferred_element_type=jnp.float32)
        + b_ref[...], 0.0)


def _tc2_body(pa_ref, pb_ref, w1_ref, b1_ref, wp_ref, bp_ref, o_ref):
    agg = pa_ref[:N] + pb_ref[:N]
    t = jnp.maximum(
        jnp.dot(agg, w1_ref[...], preferred_element_type=jnp.float32)
        + b1_ref[...], 0.0) + agg
    o_ref[...] = (jnp.dot(t, wp_ref[...], preferred_element_type=jnp.float32)
                  + bp_ref[...])


_tc1 = pl.pallas_call(
    _tc1_body,
    out_shape=jax.ShapeDtypeStruct((N, D), jnp.float32),
)

_tc2 = pl.pallas_call(
    _tc2_body,
    out_shape=jax.ShapeDtypeStruct((N, D), jnp.float32),
)


def kernel(x, edge_index, W0, b0, W1, b1, Wp, bp):
    ei = edge_index.astype(jnp.int32)
    npad = E_PAD - E
    pad_dst = N + (jnp.arange(npad, dtype=jnp.int32) % (ACC_ROWS - N))
    pad_src = jnp.arange(npad, dtype=jnp.int32) % N
    rows = jnp.concatenate([ei[0], pad_dst]).reshape(E_PAD // CH, CH)
    cols = jnp.concatenate([ei[1], pad_src]).reshape(E_PAD // CH, CH)
    zeros = jnp.zeros((ROWS_PER_TILE, D), jnp.float32)

    p0a, p0b = _sc_aggregate(x, rows, cols, zeros)
    h = _tc1(p0a, p0b, W0, b0.reshape(1, D))
    p1a, p1b = _sc_aggregate(h, rows, cols, zeros)
    return _tc2(p1a, p1b, W1, b1.reshape(1, D), Wp, bp.reshape(1, D))


# speedup vs baseline: 1.1261x; 1.0455x over previous
import functools

import jax
import jax.numpy as jnp
from jax import lax
from jax.experimental import pallas as pl
from jax.experimental.pallas import tpu as pltpu
from jax.experimental.pallas import tpu_sc as plsc

N = 10000
E = 320000
D = 128

NC = 2
NS = 16
NW = NC * NS
CH = 112
K = 90
KP = 96
G = 8
NGROUPS = 12
UNROLL = 24
KFULL = 72
E_PAD = NW * K * CH
ACC_ROWS = 10240
ROWS_PER_TILE = ACC_ROWS // NS


def _make_sc_aggregate():
    mesh = plsc.VectorSubcoreMesh(core_axis_name="c", subcore_axis_name="s")

    @functools.partial(
        pl.kernel,
        mesh=mesh,
        out_type=(jax.ShapeDtypeStruct((ACC_ROWS, D), jnp.float32),
                  jax.ShapeDtypeStruct((ACC_ROWS, D), jnp.float32)),
        scratch_types=[
            pltpu.VMEM((2 * G, CH), jnp.int32),
            pltpu.VMEM((2 * G, CH), jnp.int32),
            pltpu.VMEM((3, CH, D), jnp.float32),
            pltpu.VMEM_SHARED((ACC_ROWS, D), jnp.float32),
            [pltpu.SemaphoreType.DMA] * 3,
            [pltpu.SemaphoreType.DMA] * 3,
            pltpu.SemaphoreType.DMA,
            pltpu.SemaphoreType.DMA,
        ],
    )
    def sc_agg(src_hbm, rows_hbm, cols_hbm, zeros_hbm, part0_hbm, part1_hbm,
               rows_v, cols_v, bufs, acc, gsems, ssems, irsem, icsem):
        c = lax.axis_index("c")
        s = lax.axis_index("s")
        wid = c * NS + s

        def fire_idx(g):
            r = (g % 2) * G
            pltpu.async_copy(rows_hbm.at[wid, pl.ds(g * G, G)],
                             rows_v.at[pl.ds(r, G)], irsem)
            pltpu.async_copy(cols_hbm.at[wid, pl.ds(g * G, G)],
                             cols_v.at[pl.ds(r, G)], icsem)

        def wait_idx(g):
            r = (g % 2) * G
            pltpu.make_async_copy(rows_hbm.at[wid, pl.ds(g * G, G)],
                                  rows_v.at[pl.ds(r, G)], irsem).wait()
            pltpu.make_async_copy(cols_hbm.at[wid, pl.ds(g * G, G)],
                                  cols_v.at[pl.ds(r, G)], icsem).wait()

        def fire_gather(ring_row, b):
            pltpu.async_copy(src_hbm.at[cols_v.at[ring_row]], bufs.at[b],
                             gsems[b])

        def wait_gather(b):
            pltpu.make_async_copy(src_hbm.at[cols_v.at[0]], bufs.at[b],
                                  gsems[b]).wait()

        def fire_scatter(ring_row, b):
            pltpu.async_copy(bufs.at[b], acc.at[rows_v.at[ring_row]],
                             ssems[b], add=True)

        def wait_scatter(b):
            pltpu.make_async_copy(bufs.at[b], acc.at[rows_v.at[0]],
                                  ssems[b]).wait()

        fire_idx(0)
        wait_idx(0)
        fire_gather(0, 0)
        fire_gather(1, 1)
        pltpu.sync_copy(zeros_hbm, acc.at[pl.ds(s * ROWS_PER_TILE,
                                                ROWS_PER_TILE)])
        plsc.subcore_barrier()

        def block(m, carry):
            for q in range(UNROLL):
                j = q // G
                g = 3 * m + j
                u = q % 3
                u2 = (q + 2) % 3
                grow = ((g % 2) * G + q % G)
                wait_gather(u)
                fire_scatter(grow, u)
                if q == 0:
                    @pl.when(m >= 1)
                    def _():
                        wait_scatter(u2)
                else:
                    wait_scatter(u2)
                if q % G == 1:
                    fire_idx(g + 1)
                if q % G == 6:
                    wait_idx(g + 1)
                g2 = 3 * m + (q + 2) // G
                grow2 = (g2 % 2) * G + (q + 2) % G
                fire_gather(grow2, u2)
            return carry
        lax.fori_loop(0, KFULL // UNROLL, block, 0)
        for t in range(KFULL, K):
            g = t // G
            u = t % 3
            u2 = (t + 2) % 3
            wait_gather(u)
            fire_scatter((g % 2) * G + t % G, u)
            wait_scatter(u2)
            if t % G == 1 and g + 1 < NGROUPS:
                fire_idx(g + 1)
            if t % G == 6 and g + 1 < NGROUPS:
                wait_idx(g + 1)
            if t + 2 < K:
                g2 = (t + 2) // G
                fire_gather((g2 % 2) * G + (t + 2) % G, u2)
        wait_scatter((K - 1) % 3)
        plsc.subcore_barrier()

        sl = pl.ds(s * ROWS_PER_TILE, ROWS_PER_TILE)
        @pl.when(c == 0)
        def _():
            pltpu.sync_copy(acc.at[sl], part0_hbm.at[sl])
        @pl.when(c == 1)
        def _():
            pltpu.sync_copy(acc.at[sl], part1_hbm.at[sl])

    return sc_agg


_sc_aggregate = _make_sc_aggregate()

def _tc1_body(pa_ref, pb_ref, w_ref, b_ref, o_ref):
    agg = pa_ref[:N] + pb_ref[:N]
    o_ref[...] = jnp.maximum(
        jnp.dot(agg, w_ref[...], preferred_element_type=jnp.float32)
        + b_ref[...], 0.0)


def _tc2_body(pa_ref, pb_ref, w1_ref, b1_ref, wp_ref, bp_ref, o_ref):
    agg = pa_ref[:N] + pb_ref[:N]
    t = jnp.maximum(
        jnp.dot(agg, w1_ref[...], preferred_element_type=jnp.float32)
        + b1_ref[...], 0.0) + agg
    o_ref[...] = (jnp.dot(t, wp_ref[...], preferred_element_type=jnp.float32)
                  + bp_ref[...])


_tc1 = pl.pallas_call(
    _tc1_body,
    out_shape=jax.ShapeDtypeStruct((N, D), jnp.float32),
)

_tc2 = pl.pallas_call(
    _tc2_body,
    out_shape=jax.ShapeDtypeStruct((N, D), jnp.float32),
)


def kernel(x, edge_index, W0, b0, W1, b1, Wp, bp):
    ei = edge_index.astype(jnp.int32)
    npad = E_PAD - E
    pad_dst = N + (jnp.arange(npad, dtype=jnp.int32) % (ACC_ROWS - N))
    pad_src = jnp.arange(npad, dtype=jnp.int32) % N
    zpad = jnp.zeros((NW, KP - K, CH), jnp.int32)
    rows = jnp.concatenate(
        [jnp.concatenate([ei[0], pad_dst]).reshape(NW, K, CH), zpad], axis=1)
    cols = jnp.concatenate(
        [jnp.concatenate([ei[1], pad_src]).reshape(NW, K, CH), zpad], axis=1)
    zeros = jnp.zeros((ROWS_PER_TILE, D), jnp.float32)

    p0a, p0b = _sc_aggregate(x, rows, cols, zeros)
    h = _tc1(p0a, p0b, W0, b0.reshape(1, D))
    p1a, p1b = _sc_aggregate(h, rows, cols, zeros)
    return _tc2(p1a, p1b, W1, b1.reshape(1, D), Wp, bp.reshape(1, D))
